# baseline (speedup 1.0000x reference)
import jax
import jax.numpy as jnp
from jax.experimental import pallas as pl

_F = 100
_D = 64
_RB = 64


def _fe_kernel(mt_ref, xt_ref, o_ref):
    i = pl.program_id(0)
    x8 = xt_ref[...]
    ones = jnp.ones((1, x8.shape[1]), dtype=jnp.bfloat16)
    sub = i % 8
    for s in range(8):
        @pl.when(sub == s)
        def _(s=s):
            xa = jnp.concatenate(
                [x8[s:s + 1].astype(jnp.bfloat16), ones], axis=0
            )
            half = mt_ref[...][:, _RB * (s % 2):_RB * (s % 2) + _RB]
            o_ref[...] = jax.lax.dot_general(
                half, xa,
                (((0,), (0,)), ((), ())),
                preferred_element_type=jnp.float32,
            )


def kernel(x, emb_table, w, b):
    B, F = x.shape
    D = emb_table.shape[1]
    FD = F * D
    xt = x.T
    wt = jnp.broadcast_to(w.reshape(1, D), (F, D)).reshape(1, FD)
    tb = (emb_table + b[None, :]).reshape(1, FD)
    mt = jnp.concatenate([wt, tb], axis=0).astype(jnp.bfloat16)
    grid = (FD // _RB,)
    out_t = pl.pallas_call(
        _fe_kernel,
        grid=grid,
        in_specs=[
            pl.BlockSpec((2, 2 * _RB), lambda i: (0, i // 2)),
            pl.BlockSpec((8, B), lambda i: (i // 8, 0)),
        ],
        out_specs=pl.BlockSpec((_RB, B), lambda i: (i, 0)),
        out_shape=jax.ShapeDtypeStruct((FD, B), jnp.float32),
    )(mt, xt)
    return out_t.reshape(F, D, B).transpose(2, 0, 1)


# speedup vs baseline: 1.1710x; 1.1710x over previous
import jax
import jax.numpy as jnp
from jax.experimental import pallas as pl

_F = 100
_D = 64
_RB = 128


def _fe_kernel(mt_ref, xt_ref, o_ref):
    i = pl.program_id(0)
    x8 = xt_ref[...]
    m = mt_ref[...].reshape(3, _RB)
    ones = jnp.ones((1, x8.shape[1]), dtype=jnp.bfloat16)
    sub = i % 4
    for s in range(4):
        @pl.when(sub == s)
        def _(s=s):
            xa = jnp.concatenate(
                [x8[2 * s:2 * s + 2].astype(jnp.bfloat16), ones], axis=0
            )
            o_ref[...] = jax.lax.dot_general(
                m, xa,
                (((0,), (0,)), ((), ())),
                preferred_element_type=jnp.float32,
            )


def kernel(x, emb_table, w, b):
    B, F = x.shape
    D = emb_table.shape[1]
    FD = F * D
    xt = x.T
    emb3 = emb_table.reshape(F // 2, 2, D)
    j = jax.lax.broadcasted_iota(jnp.int32, (F // 2, 2, D), 1)
    wb = jnp.broadcast_to(w.reshape(1, 1, D), (F // 2, 2, D))
    m0 = jnp.where(j == 0, wb, 0.0)
    m1 = wb - m0
    tab = emb3 + b.reshape(1, 1, D)
    mt4 = jnp.stack([m0, m1, tab], axis=1).astype(jnp.bfloat16)
    grid = (FD // _RB,)
    out_t = pl.pallas_call(
        _fe_kernel,
        grid=grid,
        in_specs=[
            pl.BlockSpec((1, 3, 2, D), lambda i: (i, 0, 0, 0)),
            pl.BlockSpec((8, B), lambda i: (i // 4, 0)),
        ],
        out_specs=pl.BlockSpec((_RB, B), lambda i: (i, 0)),
        out_shape=jax.ShapeDtypeStruct((FD, B), jnp.float32),
    )(mt4, xt)
    return out_t.reshape(F, D, B).transpose(2, 0, 1)
